# Initial kernel scaffold; baseline (speedup 1.0000x reference)
import jax
import jax.numpy as jnp
from jax.experimental import pallas as pl


def kernel(hidden_states, encoder_hidden_states, reference_points, spatial_shapes, level_start_index, W_value, b_value, W_off, b_off, W_attn, b_attn, W_out, b_out):
    raise NotImplementedError("write your pallas kernel here")


# speedup vs baseline: 33.8323x; 33.8323x over previous
import functools

import jax
import jax.numpy as jnp
import numpy as np
from jax import lax
from jax.experimental import pallas as pl
from jax.experimental.pallas import tpu as pltpu
from jax.experimental.pallas import tpu_sc as plsc

B, Q, D, H, L, P = 8, 900, 256, 8, 4, 4
Dh = D // H
SPATIAL = np.array([[128, 128], [64, 64], [32, 32], [16, 16]], dtype=np.int64)
LVL_SIZES = [int(h * w) for h, w in SPATIAL]
LVL_STARTS = np.concatenate([[0], np.cumsum(LVL_SIZES)[:-1]]).astype(np.int64)
S = int(np.sum(LVL_SIZES))
BQ = B * Q
NLANE = H * L * P

NC, NS = 2, 16
NW = NC * NS
G8 = 8
NGRP = BQ // G8
NITER = -(-NGRP // NW)
SUB = 4
NGATH = 4 * SUB


def _linear(x, w, b, rb):
  n, k = x.shape
  m = w.shape[1]

  def kern(x_ref, w_ref, b_ref, o_ref):
    o_ref[...] = jnp.dot(x_ref[...], w_ref[...],
                         preferred_element_type=jnp.float32,
                         precision=lax.Precision.HIGHEST) + b_ref[...]

  return pl.pallas_call(
      kern,
      grid=(n // rb,),
      in_specs=[
          pl.BlockSpec((rb, k), lambda i: (i, 0)),
          pl.BlockSpec((k, m), lambda i: (0, 0)),
          pl.BlockSpec((1, m), lambda i: (0, 0)),
      ],
      out_specs=pl.BlockSpec((rb, m), lambda i: (i, 0)),
      out_shape=jax.ShapeDtypeStruct((n, m), jnp.float32),
  )(x, w, b.reshape(1, m))


_hl = np.arange(NLANE) // 16
_GRP = (_hl[:, None] == _hl[None, :]).astype(np.float32)


def _lvl_select(lvl, vals, dtype):
  out = jnp.full((1, NLANE), dtype(vals[L - 1]), dtype=dtype)
  for l in range(L - 1):
    out = jnp.where(lvl == l, dtype(vals[l]), out)
  return out


def _corner_kernel(hid, rpx, rpy, wox, box, woy, boy, wat, bat, rb):
  grp = jnp.asarray(_GRP)

  def kern(hid_ref, rpx_ref, rpy_ref, wox_ref, box_ref, woy_ref, boy_ref,
           wat_ref, bat_ref, grp_ref,
           i00, i10, i01, i11, o00, o10, o01, o11):
    pid = pl.program_id(0)
    h_ = hid_ref[...]
    offx = jnp.dot(h_, wox_ref[...], preferred_element_type=jnp.float32,
                   precision=lax.Precision.HIGHEST) + box_ref[...]
    offy = jnp.dot(h_, woy_ref[...], preferred_element_type=jnp.float32,
                   precision=lax.Precision.HIGHEST) + boy_ref[...]
    lg = jnp.dot(h_, wat_ref[...], preferred_element_type=jnp.float32,
                 precision=lax.Precision.HIGHEST) + bat_ref[...]
    mx = jnp.max(lg, axis=1, keepdims=True)
    e = jnp.exp(lg - mx)
    denom = jnp.dot(e, grp_ref[...], preferred_element_type=jnp.float32,
                    precision=lax.Precision.HIGHEST)
    attn = e / denom

    lane = lax.broadcasted_iota(jnp.int32, (1, NLANE), 1)
    lvl = (lane >> 2) & 3
    h_lane = lane >> 4
    wl = _lvl_select(lvl, [float(w) for w in SPATIAL[:, 1]], jnp.float32)
    hl = _lvl_select(lvl, [float(h) for h in SPATIAL[:, 0]], jnp.float32)
    inv_wl = _lvl_select(lvl, [1.0 / float(w) for w in SPATIAL[:, 1]],
                         jnp.float32)
    inv_hl = _lvl_select(lvl, [1.0 / float(h) for h in SPATIAL[:, 0]],
                         jnp.float32)
    wl_i = _lvl_select(lvl, [int(w) for w in SPATIAL[:, 1]], jnp.int32)
    start_i = _lvl_select(lvl, [int(s) for s in LVL_STARTS], jnp.int32)

    rbx = jnp.zeros_like(offx)
    rby = jnp.zeros_like(offy)
    for l in range(L):
      oh = (lvl == l).astype(jnp.float32)
      rbx = rbx + rpx_ref[:, l:l + 1] * oh
      rby = rby + rpy_ref[:, l:l + 1] * oh

    ux = (rbx + offx * inv_wl) * wl - 0.5
    uy = (rby + offy * inv_hl) * hl - 0.5
    x0 = jnp.floor(ux)
    y0 = jnp.floor(uy)
    x1 = x0 + 1.0
    y1 = y0 + 1.0
    vx0 = (x0 >= 0.0) & (x0 <= wl - 1.0)
    vx1 = (x1 >= 0.0) & (x1 <= wl - 1.0)
    vy0 = (y0 >= 0.0) & (y0 <= hl - 1.0)
    vy1 = (y1 >= 0.0) & (y1 <= hl - 1.0)
    ix0 = jnp.clip(x0, 0.0, wl - 1.0).astype(jnp.int32)
    ix1 = jnp.clip(x1, 0.0, wl - 1.0).astype(jnp.int32)
    iy0 = jnp.clip(y0, 0.0, hl - 1.0).astype(jnp.int32)
    iy1 = jnp.clip(y1, 0.0, hl - 1.0).astype(jnp.int32)
    wx0 = x1 - ux
    wx1 = ux - x0
    wy0 = y1 - uy
    wy1 = uy - y0

    row = pid * rb + lax.broadcasted_iota(jnp.int32, (rb, 1), 0)
    bs8 = ((row * 37284) >> 25) * (S * H)

    def emit(ix, iy, ww, vv, iref, oref):
      iref[...] = bs8 + (start_i + iy * wl_i + ix) * H + h_lane
      oref[...] = ww * vv.astype(jnp.float32) * attn

    emit(ix0, iy0, wx0 * wy0, vx0 & vy0, i00, o00)
    emit(ix1, iy0, wx1 * wy0, vx1 & vy0, i10, o10)
    emit(ix0, iy1, wx0 * wy1, vx0 & vy1, i01, o01)
    emit(ix1, iy1, wx1 * wy1, vx1 & vy1, i11, o11)

  n = hid.shape[0]
  row_spec = pl.BlockSpec((rb, NLANE), lambda i: (i, 0))
  full = lambda a: pl.BlockSpec(a.shape, lambda i: (0, 0))
  return pl.pallas_call(
      kern,
      grid=(n // rb,),
      in_specs=[
          pl.BlockSpec((rb, D), lambda i: (i, 0)),
          pl.BlockSpec((rb, L), lambda i: (i, 0)),
          pl.BlockSpec((rb, L), lambda i: (i, 0)),
          full(wox), full(box), full(woy), full(boy), full(wat), full(bat),
          full(grp),
      ],
      out_specs=[row_spec] * 8,
      out_shape=[jax.ShapeDtypeStruct((n, NLANE), jnp.int32)] * 4
      + [jax.ShapeDtypeStruct((n, NLANE), jnp.float32)] * 4,
  )(hid, rpx, rpy, wox, box, woy, boy, wat, bat, grp)


def _sc_body(table, i00, i10, i01, i11, w00, w10, w01, w11, out,
             idxv, wv, gbuf, obuf, sem):
  wid = lax.axis_index("s") * NC + lax.axis_index("c")

  def group(i, carry):
    g = wid + NW * i

    @pl.when(g < NGRP)
    def _():
      for ci, iref in enumerate((i00, i10, i01, i11)):
        pltpu.sync_copy(iref.at[g], idxv.at[pl.ds(ci * G8, G8)])
      for ci, wref in enumerate((w00, w10, w01, w11)):
        pltpu.sync_copy(wref.at[g], wv.at[pl.ds(ci * G8, G8)])

      for half in range(2):
        descs = [
            pltpu.async_copy(table.at[idxv.at[ci * G8 + half * SUB + k]],
                             gbuf.at[ci * SUB + k], sem)
            for ci in range(4) for k in range(SUB)
        ]
        for d_ in descs:
          d_.wait()

        def accum(kh, c2):
          k = kh >> 3
          h16 = (kh & 7) * 16
          acc0 = jnp.zeros((16,), jnp.float32)
          acc1 = jnp.zeros((16,), jnp.float32)
          for c in range(4):
            j = c * SUB + k
            wrow = wv[c * G8 + half * SUB + k, pl.ds(h16, 16)]
            for lp in range(16):
              wgt = wrow[lp]
              acc0 = acc0 + gbuf[j, h16 + lp, 0:16] * wgt
              acc1 = acc1 + gbuf[j, h16 + lp, 16:32] * wgt
          obuf[half * (SUB * H) + kh, 0:16] = acc0
          obuf[half * (SUB * H) + kh, 16:32] = acc1
          return c2

        lax.fori_loop(0, SUB * H, accum, 0)

      pltpu.sync_copy(obuf, out.at[pl.ds(g * (G8 * H), G8 * H)])

    return carry

  lax.fori_loop(0, NITER, group, 0)


def _sc_gather(table, i00, i10, i01, i11, w00, w10, w01, w11):
  mesh = plsc.VectorSubcoreMesh(core_axis_name="c", subcore_axis_name="s",
                                num_cores=NC, num_subcores=NS)
  fn = pl.kernel(
      _sc_body,
      out_type=jax.ShapeDtypeStruct((BQ * H, Dh), jnp.float32),
      mesh=mesh,
      scratch_types=[
          pltpu.VMEM((4 * G8, NLANE), jnp.int32),
          pltpu.VMEM((4 * G8, NLANE), jnp.float32),
          pltpu.VMEM((NGATH, NLANE, Dh), jnp.float32),
          pltpu.VMEM((G8 * H, Dh), jnp.float32),
          pltpu.SemaphoreType.DMA,
      ],
      compiler_params=pltpu.CompilerParams(use_tc_tiling_on_sc=False),
  )
  r3 = lambda a: a.reshape(NGRP, G8, NLANE)
  return fn(table, r3(i00), r3(i10), r3(i01), r3(i11),
            r3(w00), r3(w10), r3(w01), r3(w11))


def kernel(hidden_states, encoder_hidden_states, reference_points,
           spatial_shapes, level_start_index, W_value, b_value, W_off, b_off,
           W_attn, b_attn, W_out, b_out):
  enc = encoder_hidden_states.reshape(B * S, D)
  value = _linear(enc, W_value, b_value, rb=512)
  table = value.reshape(B * S * H, Dh)

  hid = hidden_states.reshape(BQ, D)
  rpx = reference_points[..., 0].reshape(BQ, L)
  rpy = reference_points[..., 1].reshape(BQ, L)
  wox = W_off[:, 0::2]
  woy = W_off[:, 1::2]
  box = b_off[0::2].reshape(1, NLANE)
  boy = b_off[1::2].reshape(1, NLANE)
  bat = b_attn.reshape(1, NLANE)

  i00, i10, i01, i11, w00, w10, w01, w11 = _corner_kernel(
      hid, rpx, rpy, wox, box, woy, boy, W_attn, bat, rb=400)

  sampled = _sc_gather(table, i00, i10, i01, i11, w00, w10, w01, w11)
  out = _linear(sampled.reshape(BQ, D), W_out, b_out, rb=480)
  return out.reshape(B, Q, D)


# speedup vs baseline: 37.4272x; 1.1063x over previous
import functools

import jax
import jax.numpy as jnp
import numpy as np
from jax import lax
from jax.experimental import pallas as pl
from jax.experimental.pallas import tpu as pltpu
from jax.experimental.pallas import tpu_sc as plsc

B, Q, D, H, L, P = 8, 900, 256, 8, 4, 4
Dh = D // H
SPATIAL = np.array([[128, 128], [64, 64], [32, 32], [16, 16]], dtype=np.int64)
LVL_SIZES = [int(h * w) for h, w in SPATIAL]
LVL_STARTS = np.concatenate([[0], np.cumsum(LVL_SIZES)[:-1]]).astype(np.int64)
S = int(np.sum(LVL_SIZES))
BQ = B * Q
NLANE = H * L * P

NC, NS = 2, 16
NW = NC * NS
G8 = 8
NGRP = BQ // G8
NITER = -(-NGRP // NW)
SUB = 2
NWAVE = G8 // SUB
NGATH = 4 * SUB


def _linear(x, w, b, rb, prec=lax.Precision.DEFAULT):
  n, k = x.shape
  m = w.shape[1]

  def kern(x_ref, w_ref, b_ref, o_ref):
    o_ref[...] = jnp.dot(x_ref[...], w_ref[...],
                         preferred_element_type=jnp.float32,
                         precision=prec) + b_ref[...]

  return pl.pallas_call(
      kern,
      grid=(n // rb,),
      in_specs=[
          pl.BlockSpec((rb, k), lambda i: (i, 0)),
          pl.BlockSpec((k, m), lambda i: (0, 0)),
          pl.BlockSpec((1, m), lambda i: (0, 0)),
      ],
      out_specs=pl.BlockSpec((rb, m), lambda i: (i, 0)),
      out_shape=jax.ShapeDtypeStruct((n, m), jnp.float32),
  )(x, w, b.reshape(1, m))


_hl = np.arange(NLANE) // 16
_GRP = (_hl[:, None] == _hl[None, :]).astype(np.float32)


def _lvl_select(lvl, vals, dtype):
  out = jnp.full((1, NLANE), dtype(vals[L - 1]), dtype=dtype)
  for l in range(L - 1):
    out = jnp.where(lvl == l, dtype(vals[l]), out)
  return out


def _corner_kernel(hid, rpx, rpy, wox, box, woy, boy, wat, bat, rb):
  grp = jnp.asarray(_GRP)

  def kern(hid_ref, rpx_ref, rpy_ref, wox_ref, box_ref, woy_ref, boy_ref,
           wat_ref, bat_ref, grp_ref,
           i00, i10, i01, i11, o00, o10, o01, o11):
    pid = pl.program_id(0)
    h_ = hid_ref[...]
    offx = jnp.dot(h_, wox_ref[...], preferred_element_type=jnp.float32,
                   precision=lax.Precision.HIGHEST) + box_ref[...]
    offy = jnp.dot(h_, woy_ref[...], preferred_element_type=jnp.float32,
                   precision=lax.Precision.HIGHEST) + boy_ref[...]
    lg = jnp.dot(h_, wat_ref[...], preferred_element_type=jnp.float32,
                 precision=lax.Precision.HIGHEST) + bat_ref[...]
    mx = jnp.max(lg, axis=1, keepdims=True)
    e = jnp.exp(lg - mx)
    denom = jnp.dot(e, grp_ref[...], preferred_element_type=jnp.float32,
                    precision=lax.Precision.HIGHEST)
    attn = e / denom

    lane = lax.broadcasted_iota(jnp.int32, (1, NLANE), 1)
    lvl = (lane >> 2) & 3
    h_lane = lane >> 4
    wl = _lvl_select(lvl, [float(w) for w in SPATIAL[:, 1]], jnp.float32)
    hl = _lvl_select(lvl, [float(h) for h in SPATIAL[:, 0]], jnp.float32)
    inv_wl = _lvl_select(lvl, [1.0 / float(w) for w in SPATIAL[:, 1]],
                         jnp.float32)
    inv_hl = _lvl_select(lvl, [1.0 / float(h) for h in SPATIAL[:, 0]],
                         jnp.float32)
    wl_i = _lvl_select(lvl, [int(w) for w in SPATIAL[:, 1]], jnp.int32)
    start_i = _lvl_select(lvl, [int(s) for s in LVL_STARTS], jnp.int32)

    rbx = jnp.zeros_like(offx)
    rby = jnp.zeros_like(offy)
    for l in range(L):
      oh = (lvl == l).astype(jnp.float32)
      rbx = rbx + rpx_ref[:, l:l + 1] * oh
      rby = rby + rpy_ref[:, l:l + 1] * oh

    ux = (rbx + offx * inv_wl) * wl - 0.5
    uy = (rby + offy * inv_hl) * hl - 0.5
    x0 = jnp.floor(ux)
    y0 = jnp.floor(uy)
    x1 = x0 + 1.0
    y1 = y0 + 1.0
    vx0 = (x0 >= 0.0) & (x0 <= wl - 1.0)
    vx1 = (x1 >= 0.0) & (x1 <= wl - 1.0)
    vy0 = (y0 >= 0.0) & (y0 <= hl - 1.0)
    vy1 = (y1 >= 0.0) & (y1 <= hl - 1.0)
    ix0 = jnp.clip(x0, 0.0, wl - 1.0).astype(jnp.int32)
    ix1 = jnp.clip(x1, 0.0, wl - 1.0).astype(jnp.int32)
    iy0 = jnp.clip(y0, 0.0, hl - 1.0).astype(jnp.int32)
    iy1 = jnp.clip(y1, 0.0, hl - 1.0).astype(jnp.int32)
    wx0 = x1 - ux
    wx1 = ux - x0
    wy0 = y1 - uy
    wy1 = uy - y0

    row = pid * rb + lax.broadcasted_iota(jnp.int32, (rb, 1), 0)
    bs8 = ((row * 37284) >> 25) * (S * H)

    def emit(ix, iy, ww, vv, iref, oref):
      iref[...] = bs8 + (start_i + iy * wl_i + ix) * H + h_lane
      oref[...] = ww * vv.astype(jnp.float32) * attn

    emit(ix0, iy0, wx0 * wy0, vx0 & vy0, i00, o00)
    emit(ix1, iy0, wx1 * wy0, vx1 & vy0, i10, o10)
    emit(ix0, iy1, wx0 * wy1, vx0 & vy1, i01, o01)
    emit(ix1, iy1, wx1 * wy1, vx1 & vy1, i11, o11)

  n = hid.shape[0]
  row_spec = pl.BlockSpec((rb, NLANE), lambda i: (i, 0))
  full = lambda a: pl.BlockSpec(a.shape, lambda i: (0, 0))
  return pl.pallas_call(
      kern,
      grid=(n // rb,),
      in_specs=[
          pl.BlockSpec((rb, D), lambda i: (i, 0)),
          pl.BlockSpec((rb, L), lambda i: (i, 0)),
          pl.BlockSpec((rb, L), lambda i: (i, 0)),
          full(wox), full(box), full(woy), full(boy), full(wat), full(bat),
          full(grp),
      ],
      out_specs=[row_spec] * 8,
      out_shape=[jax.ShapeDtypeStruct((n, NLANE), jnp.int32)] * 4
      + [jax.ShapeDtypeStruct((n, NLANE), jnp.float32)] * 4,
  )(hid, rpx, rpy, wox, box, woy, boy, wat, bat, grp)


def _sc_body(table, i00, i10, i01, i11, w00, w10, w01, w11, out,
             idxv, wv, gbuf0, gbuf1, obuf, sem0, sem1):
  wid = lax.axis_index("s") * NC + lax.axis_index("c")
  gbufs = (gbuf0, gbuf1)
  sems = (sem0, sem1)

  def fire(wave):
    buf, sem = gbufs[wave % 2], sems[wave % 2]
    return [
        pltpu.async_copy(table.at[idxv.at[ci * G8 + wave * SUB + k]],
                         buf.at[ci * SUB + k], sem)
        for ci in range(4) for k in range(SUB)
    ]

  def accum_wave(wave):
    buf = gbufs[wave % 2]

    def accum(kh, c2):
      k = kh >> 3
      h16 = (kh & 7) * 16
      acc0 = jnp.zeros((16,), jnp.float32)
      acc1 = jnp.zeros((16,), jnp.float32)
      for c in range(4):
        j = c * SUB + k
        wrow = wv[c * G8 + wave * SUB + k, pl.ds(h16, 16)]
        for lp in range(16):
          wgt = wrow[lp]
          acc0 = acc0 + buf[j, h16 + lp, 0:16] * wgt
          acc1 = acc1 + buf[j, h16 + lp, 16:32] * wgt
      obuf[wave * (SUB * H) + kh, 0:16] = acc0
      obuf[wave * (SUB * H) + kh, 16:32] = acc1
      return c2

    lax.fori_loop(0, SUB * H, accum, 0)

  def group(i, carry):
    g = wid + NW * i

    @pl.when(g < NGRP)
    def _():
      for ci, iref in enumerate((i00, i10, i01, i11)):
        pltpu.sync_copy(iref.at[g], idxv.at[pl.ds(ci * G8, G8)])
      for ci, wref in enumerate((w00, w10, w01, w11)):
        pltpu.sync_copy(wref.at[g], wv.at[pl.ds(ci * G8, G8)])

      descs = fire(0)
      for wave in range(NWAVE):
        nxt = fire(wave + 1) if wave + 1 < NWAVE else None
        for d_ in descs:
          d_.wait()
        accum_wave(wave)
        descs = nxt

      pltpu.sync_copy(obuf, out.at[pl.ds(g * (G8 * H), G8 * H)])

    return carry

  lax.fori_loop(0, NITER, group, 0)


def _sc_gather(table, i00, i10, i01, i11, w00, w10, w01, w11):
  mesh = plsc.VectorSubcoreMesh(core_axis_name="c", subcore_axis_name="s",
                                num_cores=NC, num_subcores=NS)
  fn = pl.kernel(
      _sc_body,
      out_type=jax.ShapeDtypeStruct((BQ * H, Dh), jnp.float32),
      mesh=mesh,
      scratch_types=[
          pltpu.VMEM((4 * G8, NLANE), jnp.int32),
          pltpu.VMEM((4 * G8, NLANE), jnp.float32),
          pltpu.VMEM((NGATH, NLANE, Dh), jnp.float32),
          pltpu.VMEM((NGATH, NLANE, Dh), jnp.float32),
          pltpu.VMEM((G8 * H, Dh), jnp.float32),
          pltpu.SemaphoreType.DMA,
          pltpu.SemaphoreType.DMA,
      ],
      compiler_params=pltpu.CompilerParams(use_tc_tiling_on_sc=False),
  )
  r3 = lambda a: a.reshape(NGRP, G8, NLANE)
  return fn(table, r3(i00), r3(i10), r3(i01), r3(i11),
            r3(w00), r3(w10), r3(w01), r3(w11))


def kernel(hidden_states, encoder_hidden_states, reference_points,
           spatial_shapes, level_start_index, W_value, b_value, W_off, b_off,
           W_attn, b_attn, W_out, b_out):
  enc = encoder_hidden_states.reshape(B * S, D)
  value = _linear(enc, W_value, b_value, rb=512)
  table = value.reshape(B * S * H, Dh)

  hid = hidden_states.reshape(BQ, D)
  rpx = reference_points[..., 0].reshape(BQ, L)
  rpy = reference_points[..., 1].reshape(BQ, L)
  wox = W_off[:, 0::2]
  woy = W_off[:, 1::2]
  box = b_off[0::2].reshape(1, NLANE)
  boy = b_off[1::2].reshape(1, NLANE)
  bat = b_attn.reshape(1, NLANE)

  i00, i10, i01, i11, w00, w10, w01, w11 = _corner_kernel(
      hid, rpx, rpy, wox, box, woy, boy, W_attn, bat, rb=400)

  sampled = _sc_gather(table, i00, i10, i01, i11, w00, w10, w01, w11)
  out = _linear(sampled.reshape(BQ, D), W_out, b_out, rb=480)
  return out.reshape(B, Q, D)


# speedup vs baseline: 50.6930x; 1.3544x over previous
import functools

import jax
import jax.numpy as jnp
import numpy as np
from jax import lax
from jax.experimental import pallas as pl
from jax.experimental.pallas import tpu as pltpu
from jax.experimental.pallas import tpu_sc as plsc

B, Q, D, H, L, P = 8, 900, 256, 8, 4, 4
Dh = D // H
SPATIAL = np.array([[128, 128], [64, 64], [32, 32], [16, 16]], dtype=np.int64)
LVL_SIZES = [int(h * w) for h, w in SPATIAL]
LVL_STARTS = np.concatenate([[0], np.cumsum(LVL_SIZES)[:-1]]).astype(np.int64)
S = int(np.sum(LVL_SIZES))
BQ = B * Q
NLANE = H * L * P

NC, NS = 2, 16
NW = NC * NS
G8 = 8
NGRP = BQ // G8
NITER = -(-NGRP // NW)
SUB = 2
NWAVE = G8 // SUB
NGATH = 4 * SUB


def _linear(x, w, b, rb, prec=lax.Precision.DEFAULT, out_dtype=jnp.float32):
  n, k = x.shape
  m = w.shape[1]

  def kern(x_ref, w_ref, b_ref, o_ref):
    o_ref[...] = (jnp.dot(x_ref[...], w_ref[...],
                          preferred_element_type=jnp.float32,
                          precision=prec) + b_ref[...]).astype(out_dtype)

  return pl.pallas_call(
      kern,
      grid=(n // rb,),
      in_specs=[
          pl.BlockSpec((rb, k), lambda i: (i, 0)),
          pl.BlockSpec((k, m), lambda i: (0, 0)),
          pl.BlockSpec((1, m), lambda i: (0, 0)),
      ],
      out_specs=pl.BlockSpec((rb, m), lambda i: (i, 0)),
      out_shape=jax.ShapeDtypeStruct((n, m), out_dtype),
  )(x, w, b.reshape(1, m))


_hl = np.arange(NLANE) // 16
_GRP = (_hl[:, None] == _hl[None, :]).astype(np.float32)


def _value_pack(x, wlo, whi, blo, bhi, rb):
  n, k = x.shape
  m = wlo.shape[1]

  def kern(x_ref, wlo_ref, whi_ref, blo_ref, bhi_ref, o_ref):
    x_ = x_ref[...]
    lo = jnp.dot(x_, wlo_ref[...], preferred_element_type=jnp.float32) + blo_ref[...]
    hi = jnp.dot(x_, whi_ref[...], preferred_element_type=jnp.float32) + bhi_ref[...]
    lo16 = lax.bitcast_convert_type(lo.astype(jnp.bfloat16), jnp.uint16)
    hi16 = lax.bitcast_convert_type(hi.astype(jnp.bfloat16), jnp.uint16)
    o_ref[...] = lo16.astype(jnp.uint32) | (hi16.astype(jnp.uint32) << 16)

  return pl.pallas_call(
      kern,
      grid=(n // rb,),
      in_specs=[
          pl.BlockSpec((rb, k), lambda i: (i, 0)),
          pl.BlockSpec((k, m), lambda i: (0, 0)),
          pl.BlockSpec((k, m), lambda i: (0, 0)),
          pl.BlockSpec((1, m), lambda i: (0, 0)),
          pl.BlockSpec((1, m), lambda i: (0, 0)),
      ],
      out_specs=pl.BlockSpec((rb, m), lambda i: (i, 0)),
      out_shape=jax.ShapeDtypeStruct((n, m), jnp.uint32),
  )(x, wlo, whi, blo, bhi)


def _lvl_select(lvl, vals, dtype):
  out = jnp.full((1, NLANE), dtype(vals[L - 1]), dtype=dtype)
  for l in range(L - 1):
    out = jnp.where(lvl == l, dtype(vals[l]), out)
  return out


def _corner_kernel(hid, rpx, rpy, wox, box, woy, boy, wat, bat, rb):
  grp = jnp.asarray(_GRP)

  def kern(hid_ref, rpx_ref, rpy_ref, wox_ref, box_ref, woy_ref, boy_ref,
           wat_ref, bat_ref, grp_ref,
           i00, i10, i01, i11, o00, o10, o01, o11):
    pid = pl.program_id(0)
    h_ = hid_ref[...]
    offx = jnp.dot(h_, wox_ref[...], preferred_element_type=jnp.float32,
                   precision=lax.Precision.HIGHEST) + box_ref[...]
    offy = jnp.dot(h_, woy_ref[...], preferred_element_type=jnp.float32,
                   precision=lax.Precision.HIGHEST) + boy_ref[...]
    lg = jnp.dot(h_, wat_ref[...], preferred_element_type=jnp.float32,
                 precision=lax.Precision.HIGHEST) + bat_ref[...]
    mx = jnp.max(lg, axis=1, keepdims=True)
    e = jnp.exp(lg - mx)
    denom = jnp.dot(e, grp_ref[...], preferred_element_type=jnp.float32,
                    precision=lax.Precision.HIGHEST)
    attn = e / denom

    lane = lax.broadcasted_iota(jnp.int32, (1, NLANE), 1)
    lvl = (lane >> 2) & 3
    h_lane = lane >> 4
    wl = _lvl_select(lvl, [float(w) for w in SPATIAL[:, 1]], jnp.float32)
    hl = _lvl_select(lvl, [float(h) for h in SPATIAL[:, 0]], jnp.float32)
    inv_wl = _lvl_select(lvl, [1.0 / float(w) for w in SPATIAL[:, 1]],
                         jnp.float32)
    inv_hl = _lvl_select(lvl, [1.0 / float(h) for h in SPATIAL[:, 0]],
                         jnp.float32)
    wl_i = _lvl_select(lvl, [int(w) for w in SPATIAL[:, 1]], jnp.int32)
    start_i = _lvl_select(lvl, [int(s) for s in LVL_STARTS], jnp.int32)

    rbx = jnp.zeros_like(offx)
    rby = jnp.zeros_like(offy)
    for l in range(L):
      oh = (lvl == l).astype(jnp.float32)
      rbx = rbx + rpx_ref[:, l:l + 1] * oh
      rby = rby + rpy_ref[:, l:l + 1] * oh

    ux = (rbx + offx * inv_wl) * wl - 0.5
    uy = (rby + offy * inv_hl) * hl - 0.5
    x0 = jnp.floor(ux)
    y0 = jnp.floor(uy)
    x1 = x0 + 1.0
    y1 = y0 + 1.0
    vx0 = (x0 >= 0.0) & (x0 <= wl - 1.0)
    vx1 = (x1 >= 0.0) & (x1 <= wl - 1.0)
    vy0 = (y0 >= 0.0) & (y0 <= hl - 1.0)
    vy1 = (y1 >= 0.0) & (y1 <= hl - 1.0)
    ix0 = jnp.clip(x0, 0.0, wl - 1.0).astype(jnp.int32)
    ix1 = jnp.clip(x1, 0.0, wl - 1.0).astype(jnp.int32)
    iy0 = jnp.clip(y0, 0.0, hl - 1.0).astype(jnp.int32)
    iy1 = jnp.clip(y1, 0.0, hl - 1.0).astype(jnp.int32)
    wx0 = x1 - ux
    wx1 = ux - x0
    wy0 = y1 - uy
    wy1 = uy - y0

    row = pid * rb + lax.broadcasted_iota(jnp.int32, (rb, 1), 0)
    bs8 = ((row * 37284) >> 25) * (S * H)

    def emit(ix, iy, ww, vv, iref, oref):
      iref[...] = bs8 + (start_i + iy * wl_i + ix) * H + h_lane
      oref[...] = ww * vv.astype(jnp.float32) * attn

    emit(ix0, iy0, wx0 * wy0, vx0 & vy0, i00, o00)
    emit(ix1, iy0, wx1 * wy0, vx1 & vy0, i10, o10)
    emit(ix0, iy1, wx0 * wy1, vx0 & vy1, i01, o01)
    emit(ix1, iy1, wx1 * wy1, vx1 & vy1, i11, o11)

  n = hid.shape[0]
  row_spec = pl.BlockSpec((rb, NLANE), lambda i: (i, 0))
  full = lambda a: pl.BlockSpec(a.shape, lambda i: (0, 0))
  return pl.pallas_call(
      kern,
      grid=(n // rb,),
      in_specs=[
          pl.BlockSpec((rb, D), lambda i: (i, 0)),
          pl.BlockSpec((rb, L), lambda i: (i, 0)),
          pl.BlockSpec((rb, L), lambda i: (i, 0)),
          full(wox), full(box), full(woy), full(boy), full(wat), full(bat),
          full(grp),
      ],
      out_specs=[row_spec] * 8,
      out_shape=[jax.ShapeDtypeStruct((n, NLANE), jnp.int32)] * 4
      + [jax.ShapeDtypeStruct((n, NLANE), jnp.float32)] * 4,
  )(hid, rpx, rpy, wox, box, woy, boy, wat, bat, grp)


def _sc_body(table, i00, i10, i01, i11, w00, w10, w01, w11, out,
             idxv, wv, gbuf0, gbuf1, obuf, sem0, sem1, ssem):
  wid = lax.axis_index("s") * NC + lax.axis_index("c")
  gbufs = (gbuf0, gbuf1)
  sems = (sem0, sem1)

  def fire(wave):
    buf, sem = gbufs[wave % 2], sems[wave % 2]
    return [
        pltpu.async_copy(table.at[idxv.at[ci * G8 + wave * SUB + k]],
                         buf.at[ci * SUB + k], sem)
        for ci in range(4) for k in range(SUB)
    ]

  def accum_wave(wave):
    buf = gbufs[wave % 2]

    def accum(kh, c2):
      k = kh >> 3
      h16 = (kh & 7) * 16
      acc0 = jnp.zeros((16,), jnp.float32)
      acc1 = jnp.zeros((16,), jnp.float32)
      for c in range(4):
        j = c * SUB + k
        wrow = wv[c * G8 + wave * SUB + k, pl.ds(h16, 16)]
        for lp in range(16):
          wgt = wrow[lp]
          wrd = buf[j, h16 + lp, 0:16]
          lo = plsc.bitcast(wrd << 16, jnp.float32)
          hi = plsc.bitcast(wrd & jnp.uint32(0xFFFF0000), jnp.float32)
          acc0 = acc0 + lo * wgt
          acc1 = acc1 + hi * wgt
      obuf[wave * (SUB * H) + kh, 0:16] = acc0
      obuf[wave * (SUB * H) + kh, 16:32] = acc1
      return c2

    lax.fori_loop(0, SUB * H, accum, 0)

  def group(i, carry):
    g = wid + NW * i

    @pl.when(g < NGRP)
    def _():
      stage = [
          pltpu.async_copy(ref.at[g], dst.at[pl.ds(ci * G8, G8)], ssem)
          for ci, (ref, dst) in enumerate((
              (i00, idxv), (i10, idxv), (i01, idxv), (i11, idxv)))
      ] + [
          pltpu.async_copy(ref.at[g], wv.at[pl.ds(ci * G8, G8)], ssem)
          for ci, ref in enumerate((w00, w10, w01, w11))
      ]
      for d_ in stage:
        d_.wait()

      descs = fire(0)
      for wave in range(NWAVE):
        nxt = fire(wave + 1) if wave + 1 < NWAVE else None
        for d_ in descs:
          d_.wait()
        accum_wave(wave)
        descs = nxt

      pltpu.sync_copy(obuf, out.at[pl.ds(g * (G8 * H), G8 * H)])

    return carry

  lax.fori_loop(0, NITER, group, 0)


def _sc_gather(table, i00, i10, i01, i11, w00, w10, w01, w11):
  mesh = plsc.VectorSubcoreMesh(core_axis_name="c", subcore_axis_name="s",
                                num_cores=NC, num_subcores=NS)
  fn = pl.kernel(
      _sc_body,
      out_type=jax.ShapeDtypeStruct((BQ * H, Dh), jnp.float32),
      mesh=mesh,
      scratch_types=[
          pltpu.VMEM((4 * G8, NLANE), jnp.int32),
          pltpu.VMEM((4 * G8, NLANE), jnp.float32),
          pltpu.VMEM((NGATH, NLANE, Dh // 2), jnp.uint32),
          pltpu.VMEM((NGATH, NLANE, Dh // 2), jnp.uint32),
          pltpu.VMEM((G8 * H, Dh), jnp.float32),
          pltpu.SemaphoreType.DMA,
          pltpu.SemaphoreType.DMA,
          pltpu.SemaphoreType.DMA,
      ],
      compiler_params=pltpu.CompilerParams(use_tc_tiling_on_sc=False,
                                           needs_layout_passes=False),
  )
  r3 = lambda a: a.reshape(NGRP, G8, NLANE)
  return fn(table, r3(i00), r3(i10), r3(i01), r3(i11),
            r3(w00), r3(w10), r3(w01), r3(w11))


def kernel(hidden_states, encoder_hidden_states, reference_points,
           spatial_shapes, level_start_index, W_value, b_value, W_off, b_off,
           W_attn, b_attn, W_out, b_out):
  enc = encoder_hidden_states.reshape(B * S, D)
  cols = np.arange(D).reshape(H, 2, 16)
  lo_cols = cols[:, 0, :].reshape(-1)
  hi_cols = cols[:, 1, :].reshape(-1)
  packed = _value_pack(enc, W_value[:, lo_cols], W_value[:, hi_cols],
                       b_value[lo_cols].reshape(1, -1),
                       b_value[hi_cols].reshape(1, -1), rb=512)
  table = packed.reshape(B * S * H, Dh // 2)

  hid = hidden_states.reshape(BQ, D)
  rpx = reference_points[..., 0].reshape(BQ, L)
  rpy = reference_points[..., 1].reshape(BQ, L)
  wox = W_off[:, 0::2]
  woy = W_off[:, 1::2]
  box = b_off[0::2].reshape(1, NLANE)
  boy = b_off[1::2].reshape(1, NLANE)
  bat = b_attn.reshape(1, NLANE)

  i00, i10, i01, i11, w00, w10, w01, w11 = _corner_kernel(
      hid, rpx, rpy, wox, box, woy, boy, W_attn, bat, rb=400)

  sampled = _sc_gather(table, i00, i10, i01, i11, w00, w10, w01, w11)
  out = _linear(sampled.reshape(BQ, D), W_out, b_out, rb=480)
  return out.reshape(B, Q, D)


# speedup vs baseline: 59.4094x; 1.1719x over previous
import functools

import jax
import jax.numpy as jnp
import numpy as np
from jax import lax
from jax.experimental import pallas as pl
from jax.experimental.pallas import tpu as pltpu
from jax.experimental.pallas import tpu_sc as plsc

B, Q, D, H, L, P = 8, 900, 256, 8, 4, 4
Dh = D // H
SPATIAL = np.array([[128, 128], [64, 64], [32, 32], [16, 16]], dtype=np.int64)
LVL_SIZES = [int(h * w) for h, w in SPATIAL]
LVL_STARTS = np.concatenate([[0], np.cumsum(LVL_SIZES)[:-1]]).astype(np.int64)
S = int(np.sum(LVL_SIZES))
BQ = B * Q
NLANE = H * L * P

NC, NS = 2, 16
NW = NC * NS
G8 = 8
NGRP = BQ // G8
NITER = -(-NGRP // NW)
SUB = 2
NWAVE = G8 // SUB
NGATH = 4 * SUB


def _linear(x, w, b, rb, prec=lax.Precision.DEFAULT, out_dtype=jnp.float32):
  n, k = x.shape
  m = w.shape[1]

  def kern(x_ref, w_ref, b_ref, o_ref):
    o_ref[...] = (jnp.dot(x_ref[...], w_ref[...],
                          preferred_element_type=jnp.float32,
                          precision=prec) + b_ref[...]).astype(out_dtype)

  return pl.pallas_call(
      kern,
      grid=(n // rb,),
      in_specs=[
          pl.BlockSpec((rb, k), lambda i: (i, 0)),
          pl.BlockSpec((k, m), lambda i: (0, 0)),
          pl.BlockSpec((1, m), lambda i: (0, 0)),
      ],
      out_specs=pl.BlockSpec((rb, m), lambda i: (i, 0)),
      out_shape=jax.ShapeDtypeStruct((n, m), out_dtype),
  )(x, w, b.reshape(1, m))


_hl = np.arange(NLANE) // 16
_GRP = (_hl[:, None] == _hl[None, :]).astype(np.float32)


def _value_pack(x, wlo, whi, blo, bhi, rb):
  n, k = x.shape
  m = wlo.shape[1]

  def kern(x_ref, wlo_ref, whi_ref, blo_ref, bhi_ref, o_ref):
    x_ = x_ref[...].astype(jnp.bfloat16)
    wl_ = wlo_ref[...].astype(jnp.bfloat16)
    wh_ = whi_ref[...].astype(jnp.bfloat16)
    lo = jnp.dot(x_, wl_, preferred_element_type=jnp.float32) + blo_ref[...]
    hi = jnp.dot(x_, wh_, preferred_element_type=jnp.float32) + bhi_ref[...]
    lo16 = lax.bitcast_convert_type(lo.astype(jnp.bfloat16), jnp.uint16)
    hi16 = lax.bitcast_convert_type(hi.astype(jnp.bfloat16), jnp.uint16)
    o_ref[...] = lo16.astype(jnp.uint32) | (hi16.astype(jnp.uint32) << 16)

  return pl.pallas_call(
      kern,
      grid=(n // rb,),
      in_specs=[
          pl.BlockSpec((rb, k), lambda i: (i, 0)),
          pl.BlockSpec((k, m), lambda i: (0, 0)),
          pl.BlockSpec((k, m), lambda i: (0, 0)),
          pl.BlockSpec((1, m), lambda i: (0, 0)),
          pl.BlockSpec((1, m), lambda i: (0, 0)),
      ],
      out_specs=pl.BlockSpec((rb, m), lambda i: (i, 0)),
      out_shape=jax.ShapeDtypeStruct((n, m), jnp.uint32),
  )(x, wlo, whi, blo, bhi)


def _lvl_select(lvl, vals, dtype):
  out = jnp.full((1, NLANE), dtype(vals[L - 1]), dtype=dtype)
  for l in range(L - 1):
    out = jnp.where(lvl == l, dtype(vals[l]), out)
  return out


def _corner_kernel(hid, rpx, rpy, wox, box, woy, boy, wat, bat, rb):
  grp = jnp.asarray(_GRP)

  def kern(hid_ref, rpx_ref, rpy_ref, wox_ref, box_ref, woy_ref, boy_ref,
           wat_ref, bat_ref, grp_ref, iref, oref):
    pid = pl.program_id(0)
    h_ = hid_ref[...]
    offx = jnp.dot(h_, wox_ref[...], preferred_element_type=jnp.float32,
                   precision=lax.Precision.DEFAULT) + box_ref[...]
    offy = jnp.dot(h_, woy_ref[...], preferred_element_type=jnp.float32,
                   precision=lax.Precision.DEFAULT) + boy_ref[...]
    lg = jnp.dot(h_, wat_ref[...], preferred_element_type=jnp.float32,
                 precision=lax.Precision.DEFAULT) + bat_ref[...]
    mx = jnp.max(lg, axis=1, keepdims=True)
    e = jnp.exp(lg - mx)
    denom = jnp.dot(e, grp_ref[...], preferred_element_type=jnp.float32,
                    precision=lax.Precision.DEFAULT)
    attn = e / denom

    lane = lax.broadcasted_iota(jnp.int32, (1, NLANE), 1)
    lvl = (lane >> 2) & 3
    h_lane = lane >> 4
    wl = _lvl_select(lvl, [float(w) for w in SPATIAL[:, 1]], jnp.float32)
    hl = _lvl_select(lvl, [float(h) for h in SPATIAL[:, 0]], jnp.float32)
    inv_wl = _lvl_select(lvl, [1.0 / float(w) for w in SPATIAL[:, 1]],
                         jnp.float32)
    inv_hl = _lvl_select(lvl, [1.0 / float(h) for h in SPATIAL[:, 0]],
                         jnp.float32)
    wl_i = _lvl_select(lvl, [int(w) for w in SPATIAL[:, 1]], jnp.int32)
    start_i = _lvl_select(lvl, [int(s) for s in LVL_STARTS], jnp.int32)

    rbx = jnp.zeros_like(offx)
    rby = jnp.zeros_like(offy)
    for l in range(L):
      oh = (lvl == l).astype(jnp.float32)
      rbx = rbx + rpx_ref[:, l:l + 1] * oh
      rby = rby + rpy_ref[:, l:l + 1] * oh

    ux = (rbx + offx * inv_wl) * wl - 0.5
    uy = (rby + offy * inv_hl) * hl - 0.5
    x0 = jnp.floor(ux)
    y0 = jnp.floor(uy)
    x1 = x0 + 1.0
    y1 = y0 + 1.0
    vx0 = (x0 >= 0.0) & (x0 <= wl - 1.0)
    vx1 = (x1 >= 0.0) & (x1 <= wl - 1.0)
    vy0 = (y0 >= 0.0) & (y0 <= hl - 1.0)
    vy1 = (y1 >= 0.0) & (y1 <= hl - 1.0)
    ix0 = jnp.clip(x0, 0.0, wl - 1.0).astype(jnp.int32)
    ix1 = jnp.clip(x1, 0.0, wl - 1.0).astype(jnp.int32)
    iy0 = jnp.clip(y0, 0.0, hl - 1.0).astype(jnp.int32)
    iy1 = jnp.clip(y1, 0.0, hl - 1.0).astype(jnp.int32)
    wx0 = x1 - ux
    wx1 = ux - x0
    wy0 = y1 - uy
    wy1 = uy - y0

    row = pid * rb + lax.broadcasted_iota(jnp.int32, (rb, 1), 0)
    bs8 = ((row * 37284) >> 25) * (S * H)

    def emit(ci, ix, iy, ww, vv):
      iref[:, ci * NLANE:(ci + 1) * NLANE] = (
          bs8 + (start_i + iy * wl_i + ix) * H + h_lane)
      oref[:, ci * NLANE:(ci + 1) * NLANE] = ww * vv.astype(jnp.float32) * attn

    emit(0, ix0, iy0, wx0 * wy0, vx0 & vy0)
    emit(1, ix1, iy0, wx1 * wy0, vx1 & vy0)
    emit(2, ix0, iy1, wx0 * wy1, vx0 & vy1)
    emit(3, ix1, iy1, wx1 * wy1, vx1 & vy1)

  n = hid.shape[0]
  row_spec = pl.BlockSpec((rb, 4 * NLANE), lambda i: (i, 0))
  full = lambda a: pl.BlockSpec(a.shape, lambda i: (0, 0))
  return pl.pallas_call(
      kern,
      grid=(n // rb,),
      in_specs=[
          pl.BlockSpec((rb, D), lambda i: (i, 0)),
          pl.BlockSpec((rb, L), lambda i: (i, 0)),
          pl.BlockSpec((rb, L), lambda i: (i, 0)),
          full(wox), full(box), full(woy), full(boy), full(wat), full(bat),
          full(grp),
      ],
      out_specs=[row_spec] * 2,
      out_shape=[jax.ShapeDtypeStruct((n, 4 * NLANE), jnp.int32),
                 jax.ShapeDtypeStruct((n, 4 * NLANE), jnp.float32)],
  )(hid, rpx, rpy, wox, box, woy, boy, wat, bat, grp)


def _sc_body(table, i_all, w_all, out,
             idxv, wv, gbuf0, gbuf1, obuf, sem0, sem1, ssem):
  wid = lax.axis_index("s") * NC + lax.axis_index("c")
  gbufs = (gbuf0, gbuf1)
  sems = (sem0, sem1)

  def fire(wave):
    buf, sem = gbufs[wave % 2], sems[wave % 2]
    return [
        pltpu.async_copy(
            table.at[idxv.at[wave * SUB + k, pl.ds(ci * NLANE, NLANE)]],
            buf.at[ci * SUB + k], sem)
        for ci in range(4) for k in range(SUB)
    ]

  def accum_wave(wave):
    buf = gbufs[wave % 2]

    def accum(kh, c2):
      k = kh >> 3
      h16 = (kh & 7) * 16
      acc0 = jnp.zeros((16,), jnp.float32)
      acc1 = jnp.zeros((16,), jnp.float32)
      for c in range(4):
        j = c * SUB + k
        wrow = wv[wave * SUB + k, pl.ds(c * NLANE + h16, 16)]
        for lp in range(16):
          wgt = wrow[lp]
          wrd = buf[j, h16 + lp, 0:16]
          lo = plsc.bitcast(wrd << 16, jnp.float32)
          hi = plsc.bitcast(wrd & jnp.uint32(0xFFFF0000), jnp.float32)
          acc0 = acc0 + lo * wgt
          acc1 = acc1 + hi * wgt
      obuf[wave * (SUB * H) + kh, 0:16] = acc0
      obuf[wave * (SUB * H) + kh, 16:32] = acc1
      return c2

    lax.fori_loop(0, SUB * H, accum, 0)

  def group(i, carry):
    g = wid + NW * i

    @pl.when(g < NGRP)
    def _():
      s1 = pltpu.async_copy(i_all.at[g], idxv, ssem)
      s2 = pltpu.async_copy(w_all.at[g], wv, ssem)
      s1.wait()
      s2.wait()

      descs = fire(0)
      for wave in range(NWAVE):
        nxt = fire(wave + 1) if wave + 1 < NWAVE else None
        for d_ in descs:
          d_.wait()
        accum_wave(wave)
        descs = nxt

      pltpu.sync_copy(obuf, out.at[pl.ds(g * (G8 * H), G8 * H)])

    return carry

  lax.fori_loop(0, NITER, group, 0)


def _sc_gather(table, i_all, w_all):
  mesh = plsc.VectorSubcoreMesh(core_axis_name="c", subcore_axis_name="s",
                                num_cores=NC, num_subcores=NS)
  fn = pl.kernel(
      _sc_body,
      out_type=jax.ShapeDtypeStruct((BQ * H, Dh), jnp.float32),
      mesh=mesh,
      scratch_types=[
          pltpu.VMEM((G8, 4 * NLANE), jnp.int32),
          pltpu.VMEM((G8, 4 * NLANE), jnp.float32),
          pltpu.VMEM((NGATH, NLANE, Dh // 2), jnp.uint32),
          pltpu.VMEM((NGATH, NLANE, Dh // 2), jnp.uint32),
          pltpu.VMEM((G8 * H, Dh), jnp.float32),
          pltpu.SemaphoreType.DMA,
          pltpu.SemaphoreType.DMA,
          pltpu.SemaphoreType.DMA,
      ],
      compiler_params=pltpu.CompilerParams(use_tc_tiling_on_sc=False,
                                           needs_layout_passes=False),
  )
  r3 = lambda a: a.reshape(NGRP, G8, 4 * NLANE)
  return fn(table, r3(i_all), r3(w_all))


def kernel(hidden_states, encoder_hidden_states, reference_points,
           spatial_shapes, level_start_index, W_value, b_value, W_off, b_off,
           W_attn, b_attn, W_out, b_out):
  enc = encoder_hidden_states.reshape(B * S, D)
  cols = np.arange(D).reshape(H, 2, 16)
  lo_cols = cols[:, 0, :].reshape(-1)
  hi_cols = cols[:, 1, :].reshape(-1)
  packed = _value_pack(enc, W_value[:, lo_cols], W_value[:, hi_cols],
                       b_value[lo_cols].reshape(1, -1),
                       b_value[hi_cols].reshape(1, -1), rb=1088)
  table = packed.reshape(B * S * H, Dh // 2)

  hid = hidden_states.reshape(BQ, D)
  rpx = reference_points[..., 0].reshape(BQ, L)
  rpy = reference_points[..., 1].reshape(BQ, L)
  wox = W_off[:, 0::2]
  woy = W_off[:, 1::2]
  box = b_off[0::2].reshape(1, NLANE)
  boy = b_off[1::2].reshape(1, NLANE)
  bat = b_attn.reshape(1, NLANE)

  i_all, w_all = _corner_kernel(
      hid, rpx, rpy, wox, box, woy, boy, W_attn, bat, rb=720)

  sampled = _sc_gather(table, i_all, w_all)
  out = _linear(sampled.reshape(BQ, D), W_out, b_out, rb=480)
  return out.reshape(B, Q, D)


# speedup vs baseline: 65.9035x; 1.1093x over previous
import functools

import jax
import jax.numpy as jnp
import numpy as np
from jax import lax
from jax.experimental import pallas as pl
from jax.experimental.pallas import tpu as pltpu
from jax.experimental.pallas import tpu_sc as plsc

B, Q, D, H, L, P = 8, 900, 256, 8, 4, 4
Dh = D // H
SPATIAL = np.array([[128, 128], [64, 64], [32, 32], [16, 16]], dtype=np.int64)
LVL_SIZES = [int(h * w) for h, w in SPATIAL]
LVL_STARTS = np.concatenate([[0], np.cumsum(LVL_SIZES)[:-1]]).astype(np.int64)
S = int(np.sum(LVL_SIZES))
BQ = B * Q
NLANE = H * L * P

NC, NS = 2, 16
NW = NC * NS
G8 = 8
NGRP = BQ // G8
NITER = -(-NGRP // NW)
SUB = 2
NWAVE = G8 // SUB
NGATH = 4 * SUB


def _linear(x, w, b, rb, prec=lax.Precision.DEFAULT, out_dtype=jnp.float32):
  n, k = x.shape
  m = w.shape[1]

  def kern(x_ref, w_ref, b_ref, o_ref):
    o_ref[...] = (jnp.dot(x_ref[...], w_ref[...],
                          preferred_element_type=jnp.float32,
                          precision=prec) + b_ref[...]).astype(out_dtype)

  return pl.pallas_call(
      kern,
      grid=(n // rb,),
      in_specs=[
          pl.BlockSpec((rb, k), lambda i: (i, 0)),
          pl.BlockSpec((k, m), lambda i: (0, 0)),
          pl.BlockSpec((1, m), lambda i: (0, 0)),
      ],
      out_specs=pl.BlockSpec((rb, m), lambda i: (i, 0)),
      out_shape=jax.ShapeDtypeStruct((n, m), out_dtype),
  )(x, w, b.reshape(1, m))


_hl = np.arange(NLANE) // 16
_GRP = (_hl[:, None] == _hl[None, :]).astype(np.float32)


def _value_pack(x, wlo, whi, blo, bhi, rb):
  n, k = x.shape
  m = wlo.shape[1]

  def kern(x_ref, wlo_ref, whi_ref, blo_ref, bhi_ref, o_ref):
    x_ = x_ref[...].astype(jnp.bfloat16)
    wl_ = wlo_ref[...].astype(jnp.bfloat16)
    wh_ = whi_ref[...].astype(jnp.bfloat16)
    lo = jnp.dot(x_, wl_, preferred_element_type=jnp.float32) + blo_ref[...]
    hi = jnp.dot(x_, wh_, preferred_element_type=jnp.float32) + bhi_ref[...]
    lo16 = lax.bitcast_convert_type(lo.astype(jnp.bfloat16), jnp.uint16)
    hi16 = lax.bitcast_convert_type(hi.astype(jnp.bfloat16), jnp.uint16)
    o_ref[...] = lo16.astype(jnp.uint32) | (hi16.astype(jnp.uint32) << 16)

  return pl.pallas_call(
      kern,
      grid=(n // rb,),
      in_specs=[
          pl.BlockSpec((rb, k), lambda i: (i, 0)),
          pl.BlockSpec((k, m), lambda i: (0, 0)),
          pl.BlockSpec((k, m), lambda i: (0, 0)),
          pl.BlockSpec((1, m), lambda i: (0, 0)),
          pl.BlockSpec((1, m), lambda i: (0, 0)),
      ],
      out_specs=pl.BlockSpec((rb, m), lambda i: (i, 0)),
      out_shape=jax.ShapeDtypeStruct((n, m), jnp.uint32),
  )(x, wlo, whi, blo, bhi)


def _lvl_select(lvl, vals, dtype):
  out = jnp.full((1, NLANE), dtype(vals[L - 1]), dtype=dtype)
  for l in range(L - 1):
    out = jnp.where(lvl == l, dtype(vals[l]), out)
  return out


def _corner_kernel(hid, rpx, rpy, wox, box, woy, boy, wat, bat, rb):
  grp = jnp.asarray(_GRP)

  def kern(hid_ref, rpx_ref, rpy_ref, wox_ref, box_ref, woy_ref, boy_ref,
           wat_ref, bat_ref, grp_ref, iref, oref):
    pid = pl.program_id(0)
    h_ = hid_ref[...]
    offx = jnp.dot(h_, wox_ref[...], preferred_element_type=jnp.float32,
                   precision=lax.Precision.DEFAULT) + box_ref[...]
    offy = jnp.dot(h_, woy_ref[...], preferred_element_type=jnp.float32,
                   precision=lax.Precision.DEFAULT) + boy_ref[...]
    lg = jnp.dot(h_, wat_ref[...], preferred_element_type=jnp.float32,
                 precision=lax.Precision.DEFAULT) + bat_ref[...]
    mx = jnp.max(lg, axis=1, keepdims=True)
    e = jnp.exp(lg - mx)
    denom = jnp.dot(e, grp_ref[...], preferred_element_type=jnp.float32,
                    precision=lax.Precision.DEFAULT)
    attn = e / denom

    lane = lax.broadcasted_iota(jnp.int32, (1, NLANE), 1)
    lvl = (lane >> 2) & 3
    h_lane = lane >> 4
    wl = _lvl_select(lvl, [float(w) for w in SPATIAL[:, 1]], jnp.float32)
    hl = _lvl_select(lvl, [float(h) for h in SPATIAL[:, 0]], jnp.float32)
    inv_wl = _lvl_select(lvl, [1.0 / float(w) for w in SPATIAL[:, 1]],
                         jnp.float32)
    inv_hl = _lvl_select(lvl, [1.0 / float(h) for h in SPATIAL[:, 0]],
                         jnp.float32)
    wl_i = _lvl_select(lvl, [int(w) for w in SPATIAL[:, 1]], jnp.int32)
    start_i = _lvl_select(lvl, [int(s) for s in LVL_STARTS], jnp.int32)

    rbx = jnp.zeros_like(offx)
    rby = jnp.zeros_like(offy)
    for l in range(L):
      oh = (lvl == l).astype(jnp.float32)
      rbx = rbx + rpx_ref[:, l:l + 1] * oh
      rby = rby + rpy_ref[:, l:l + 1] * oh

    ux = (rbx + offx * inv_wl) * wl - 0.5
    uy = (rby + offy * inv_hl) * hl - 0.5
    x0 = jnp.floor(ux)
    y0 = jnp.floor(uy)
    x1 = x0 + 1.0
    y1 = y0 + 1.0
    vx0 = (x0 >= 0.0) & (x0 <= wl - 1.0)
    vx1 = (x1 >= 0.0) & (x1 <= wl - 1.0)
    vy0 = (y0 >= 0.0) & (y0 <= hl - 1.0)
    vy1 = (y1 >= 0.0) & (y1 <= hl - 1.0)
    ix0 = jnp.clip(x0, 0.0, wl - 1.0).astype(jnp.int32)
    ix1 = jnp.clip(x1, 0.0, wl - 1.0).astype(jnp.int32)
    iy0 = jnp.clip(y0, 0.0, hl - 1.0).astype(jnp.int32)
    iy1 = jnp.clip(y1, 0.0, hl - 1.0).astype(jnp.int32)
    wx0 = x1 - ux
    wx1 = ux - x0
    wy0 = y1 - uy
    wy1 = uy - y0

    row = pid * rb + lax.broadcasted_iota(jnp.int32, (rb, 1), 0)
    bs8 = ((row * 37284) >> 25) * (S * H)

    ng = rb // G8

    def emit(ci, ix, iy, ww, vv):
      iref[:, ci] = (bs8 + (start_i + iy * wl_i + ix) * H
                     + h_lane).reshape(ng, G8, NLANE)
      oref[:, ci] = (ww * vv.astype(jnp.float32) * attn).reshape(ng, G8, NLANE)

    emit(0, ix0, iy0, wx0 * wy0, vx0 & vy0)
    emit(1, ix1, iy0, wx1 * wy0, vx1 & vy0)
    emit(2, ix0, iy1, wx0 * wy1, vx0 & vy1)
    emit(3, ix1, iy1, wx1 * wy1, vx1 & vy1)

  n = hid.shape[0]
  ng = rb // G8
  row_spec = pl.BlockSpec((ng, 4, G8, NLANE), lambda i: (i, 0, 0, 0))
  full = lambda a: pl.BlockSpec(a.shape, lambda i: (0, 0))
  return pl.pallas_call(
      kern,
      grid=(n // rb,),
      in_specs=[
          pl.BlockSpec((rb, D), lambda i: (i, 0)),
          pl.BlockSpec((rb, L), lambda i: (i, 0)),
          pl.BlockSpec((rb, L), lambda i: (i, 0)),
          full(wox), full(box), full(woy), full(boy), full(wat), full(bat),
          full(grp),
      ],
      out_specs=[row_spec] * 2,
      out_shape=[jax.ShapeDtypeStruct((n // G8, 4, G8, NLANE), jnp.int32),
                 jax.ShapeDtypeStruct((n // G8, 4, G8, NLANE), jnp.float32)],
  )(hid, rpx, rpy, wox, box, woy, boy, wat, bat, grp)


def _sc_body(table, i_all, w_all, out,
             idxv, wv, gbuf0, gbuf1, obuf, sem0, sem1, ssem):
  wid = lax.axis_index("s") * NC + lax.axis_index("c")
  gbufs = (gbuf0, gbuf1)
  sems = (sem0, sem1)

  def fire(wave):
    buf, sem = gbufs[wave % 2], sems[wave % 2]
    return [
        pltpu.async_copy(
            table.at[idxv.at[ci, wave * SUB + k]],
            buf.at[ci * SUB + k], sem)
        for ci in range(4) for k in range(SUB)
    ]

  def accum_wave(wave):
    buf = gbufs[wave % 2]

    def accum(kh, c2):
      k = kh >> 3
      h16 = (kh & 7) * 16
      acc0 = jnp.zeros((16,), jnp.float32)
      acc1 = jnp.zeros((16,), jnp.float32)
      for c in range(4):
        j = c * SUB + k
        wrow = wv[c, wave * SUB + k, pl.ds(h16, 16)]
        for lp in range(16):
          wgt = wrow[lp]
          wrd = buf[j, h16 + lp, 0:16]
          lo = plsc.bitcast(wrd << 16, jnp.float32)
          hi = plsc.bitcast(wrd & jnp.uint32(0xFFFF0000), jnp.float32)
          acc0 = acc0 + lo * wgt
          acc1 = acc1 + hi * wgt
      obuf[wave * (SUB * H) + kh, 0:16] = acc0
      obuf[wave * (SUB * H) + kh, 16:32] = acc1
      return c2

    lax.fori_loop(0, SUB * H, accum, 0)

  def group(i, carry):
    g = wid + NW * i

    @pl.when(g < NGRP)
    def _():
      s1 = pltpu.async_copy(i_all.at[g], idxv, ssem)
      s2 = pltpu.async_copy(w_all.at[g], wv, ssem)
      s1.wait()
      s2.wait()

      descs = fire(0)
      for wave in range(NWAVE):
        nxt = fire(wave + 1) if wave + 1 < NWAVE else None
        for d_ in descs:
          d_.wait()
        accum_wave(wave)
        descs = nxt

      pltpu.sync_copy(obuf, out.at[pl.ds(g * (G8 * H), G8 * H)])

    return carry

  lax.fori_loop(0, NITER, group, 0)


def _sc_gather(table, i_all, w_all):
  mesh = plsc.VectorSubcoreMesh(core_axis_name="c", subcore_axis_name="s",
                                num_cores=NC, num_subcores=NS)
  fn = pl.kernel(
      _sc_body,
      out_type=jax.ShapeDtypeStruct((BQ * H, Dh), jnp.float32),
      mesh=mesh,
      scratch_types=[
          pltpu.VMEM((4, G8, NLANE), jnp.int32),
          pltpu.VMEM((4, G8, NLANE), jnp.float32),
          pltpu.VMEM((NGATH, NLANE, Dh // 2), jnp.uint32),
          pltpu.VMEM((NGATH, NLANE, Dh // 2), jnp.uint32),
          pltpu.VMEM((G8 * H, Dh), jnp.float32),
          pltpu.SemaphoreType.DMA,
          pltpu.SemaphoreType.DMA,
          pltpu.SemaphoreType.DMA,
      ],
      compiler_params=pltpu.CompilerParams(use_tc_tiling_on_sc=False,
                                           needs_layout_passes=False),
  )
  return fn(table, i_all, w_all)


def kernel(hidden_states, encoder_hidden_states, reference_points,
           spatial_shapes, level_start_index, W_value, b_value, W_off, b_off,
           W_attn, b_attn, W_out, b_out):
  enc = encoder_hidden_states.reshape(B * S, D)
  cols = np.arange(D).reshape(H, 2, 16)
  lo_cols = cols[:, 0, :].reshape(-1)
  hi_cols = cols[:, 1, :].reshape(-1)
  packed = _value_pack(enc, W_value[:, lo_cols], W_value[:, hi_cols],
                       b_value[lo_cols].reshape(1, -1),
                       b_value[hi_cols].reshape(1, -1), rb=2176)
  table = packed.reshape(B * S * H, Dh // 2)

  hid = hidden_states.reshape(BQ, D)
  rpx = reference_points[..., 0].reshape(BQ, L)
  rpy = reference_points[..., 1].reshape(BQ, L)
  wox = W_off[:, 0::2]
  woy = W_off[:, 1::2]
  box = b_off[0::2].reshape(1, NLANE)
  boy = b_off[1::2].reshape(1, NLANE)
  bat = b_attn.reshape(1, NLANE)

  i_all, w_all = _corner_kernel(
      hid, rpx, rpy, wox, box, woy, boy, W_attn, bat, rb=720)

  sampled = _sc_gather(table, i_all, w_all)
  out = _linear(sampled.reshape(BQ, D), W_out, b_out, rb=480)
  return out.reshape(B, Q, D)


# speedup vs baseline: 75.2085x; 1.1412x over previous
import functools

import jax
import jax.numpy as jnp
import numpy as np
from jax import lax
from jax.experimental import pallas as pl
from jax.experimental.pallas import tpu as pltpu
from jax.experimental.pallas import tpu_sc as plsc

B, Q, D, H, L, P = 8, 900, 256, 8, 4, 4
Dh = D // H
SPATIAL = np.array([[128, 128], [64, 64], [32, 32], [16, 16]], dtype=np.int64)
LVL_SIZES = [int(h * w) for h, w in SPATIAL]
LVL_STARTS = np.concatenate([[0], np.cumsum(LVL_SIZES)[:-1]]).astype(np.int64)
S = int(np.sum(LVL_SIZES))
BQ = B * Q
NLANE = H * L * P

NC, NS = 2, 16
NW = NC * NS
G8 = 8
NGRP = BQ // G8
NITER = -(-NGRP // NW)
SUB = 4
NWAVE = G8 // SUB
NGATH = 4 * SUB


def _linear(x, w, b, rb, prec=lax.Precision.DEFAULT, out_dtype=jnp.float32):
  n, k = x.shape
  m = w.shape[1]

  def kern(x_ref, w_ref, b_ref, o_ref):
    o_ref[...] = (jnp.dot(x_ref[...], w_ref[...],
                          preferred_element_type=jnp.float32,
                          precision=prec) + b_ref[...]).astype(out_dtype)

  return pl.pallas_call(
      kern,
      grid=(n // rb,),
      in_specs=[
          pl.BlockSpec((rb, k), lambda i: (i, 0)),
          pl.BlockSpec((k, m), lambda i: (0, 0)),
          pl.BlockSpec((1, m), lambda i: (0, 0)),
      ],
      out_specs=pl.BlockSpec((rb, m), lambda i: (i, 0)),
      out_shape=jax.ShapeDtypeStruct((n, m), out_dtype),
  )(x, w, b.reshape(1, m))


_hl = np.arange(NLANE) // 16
_GRP = (_hl[:, None] == _hl[None, :]).astype(np.float32)


def _value_pack(x, wlo, whi, blo, bhi, rb):
  n, k = x.shape
  m = wlo.shape[1]

  def kern(x_ref, wlo_ref, whi_ref, blo_ref, bhi_ref, o_ref):
    x_ = x_ref[...].astype(jnp.bfloat16)
    wl_ = wlo_ref[...].astype(jnp.bfloat16)
    wh_ = whi_ref[...].astype(jnp.bfloat16)
    lo = jnp.dot(x_, wl_, preferred_element_type=jnp.float32) + blo_ref[...]
    hi = jnp.dot(x_, wh_, preferred_element_type=jnp.float32) + bhi_ref[...]
    lo16 = lax.bitcast_convert_type(lo.astype(jnp.bfloat16), jnp.uint16)
    hi16 = lax.bitcast_convert_type(hi.astype(jnp.bfloat16), jnp.uint16)
    o_ref[...] = lo16.astype(jnp.uint32) | (hi16.astype(jnp.uint32) << 16)

  return pl.pallas_call(
      kern,
      grid=(n // rb,),
      in_specs=[
          pl.BlockSpec((rb, k), lambda i: (i, 0)),
          pl.BlockSpec((k, m), lambda i: (0, 0)),
          pl.BlockSpec((k, m), lambda i: (0, 0)),
          pl.BlockSpec((1, m), lambda i: (0, 0)),
          pl.BlockSpec((1, m), lambda i: (0, 0)),
      ],
      out_specs=pl.BlockSpec((rb, m), lambda i: (i, 0)),
      out_shape=jax.ShapeDtypeStruct((n, m), jnp.uint32),
  )(x, wlo, whi, blo, bhi)


def _lvl_select(lvl, vals, dtype):
  out = jnp.full((1, NLANE), dtype(vals[L - 1]), dtype=dtype)
  for l in range(L - 1):
    out = jnp.where(lvl == l, dtype(vals[l]), out)
  return out


def _corner_kernel(hid, rpx, rpy, wox, box, woy, boy, wat, bat, rb):
  grp = jnp.asarray(_GRP)

  def kern(hid_ref, rpx_ref, rpy_ref, wox_ref, box_ref, woy_ref, boy_ref,
           wat_ref, bat_ref, grp_ref, iref, oref):
    pid = pl.program_id(0)
    h_ = hid_ref[...]
    offx = jnp.dot(h_, wox_ref[...], preferred_element_type=jnp.float32,
                   precision=lax.Precision.DEFAULT) + box_ref[...]
    offy = jnp.dot(h_, woy_ref[...], preferred_element_type=jnp.float32,
                   precision=lax.Precision.DEFAULT) + boy_ref[...]
    lg = jnp.dot(h_, wat_ref[...], preferred_element_type=jnp.float32,
                 precision=lax.Precision.DEFAULT) + bat_ref[...]
    mx = jnp.max(lg, axis=1, keepdims=True)
    e = jnp.exp(lg - mx)
    denom = jnp.dot(e, grp_ref[...], preferred_element_type=jnp.float32,
                    precision=lax.Precision.DEFAULT)
    attn = e / denom

    lane = lax.broadcasted_iota(jnp.int32, (1, NLANE), 1)
    lvl = (lane >> 2) & 3
    h_lane = lane >> 4
    wl = _lvl_select(lvl, [float(w) for w in SPATIAL[:, 1]], jnp.float32)
    hl = _lvl_select(lvl, [float(h) for h in SPATIAL[:, 0]], jnp.float32)
    inv_wl = _lvl_select(lvl, [1.0 / float(w) for w in SPATIAL[:, 1]],
                         jnp.float32)
    inv_hl = _lvl_select(lvl, [1.0 / float(h) for h in SPATIAL[:, 0]],
                         jnp.float32)
    wl_i = _lvl_select(lvl, [int(w) for w in SPATIAL[:, 1]], jnp.int32)
    start_i = _lvl_select(lvl, [int(s) for s in LVL_STARTS], jnp.int32)

    rbx = jnp.zeros_like(offx)
    rby = jnp.zeros_like(offy)
    for l in range(L):
      oh = (lvl == l).astype(jnp.float32)
      rbx = rbx + rpx_ref[:, l:l + 1] * oh
      rby = rby + rpy_ref[:, l:l + 1] * oh

    ux = (rbx + offx * inv_wl) * wl - 0.5
    uy = (rby + offy * inv_hl) * hl - 0.5
    x0 = jnp.floor(ux)
    y0 = jnp.floor(uy)
    x1 = x0 + 1.0
    y1 = y0 + 1.0
    vx0 = (x0 >= 0.0) & (x0 <= wl - 1.0)
    vx1 = (x1 >= 0.0) & (x1 <= wl - 1.0)
    vy0 = (y0 >= 0.0) & (y0 <= hl - 1.0)
    vy1 = (y1 >= 0.0) & (y1 <= hl - 1.0)
    ix0 = jnp.clip(x0, 0.0, wl - 1.0).astype(jnp.int32)
    ix1 = jnp.clip(x1, 0.0, wl - 1.0).astype(jnp.int32)
    iy0 = jnp.clip(y0, 0.0, hl - 1.0).astype(jnp.int32)
    iy1 = jnp.clip(y1, 0.0, hl - 1.0).astype(jnp.int32)
    wx0 = x1 - ux
    wx1 = ux - x0
    wy0 = y1 - uy
    wy1 = uy - y0

    row = pid * rb + lax.broadcasted_iota(jnp.int32, (rb, 1), 0)
    bs8 = ((row * 37284) >> 25) * (S * H)

    ng = rb // G8

    def emit(ci, ix, iy, ww, vv):
      iref[:, ci] = (bs8 + (start_i + iy * wl_i + ix) * H
                     + h_lane).reshape(ng, G8, NLANE)
      oref[:, ci] = (ww * vv.astype(jnp.float32) * attn).reshape(ng, G8, NLANE)

    emit(0, ix0, iy0, wx0 * wy0, vx0 & vy0)
    emit(1, ix1, iy0, wx1 * wy0, vx1 & vy0)
    emit(2, ix0, iy1, wx0 * wy1, vx0 & vy1)
    emit(3, ix1, iy1, wx1 * wy1, vx1 & vy1)

  n = hid.shape[0]
  ng = rb // G8
  row_spec = pl.BlockSpec((ng, 4, G8, NLANE), lambda i: (i, 0, 0, 0))
  full = lambda a: pl.BlockSpec(a.shape, lambda i: (0, 0))
  return pl.pallas_call(
      kern,
      grid=(n // rb,),
      in_specs=[
          pl.BlockSpec((rb, D), lambda i: (i, 0)),
          pl.BlockSpec((rb, L), lambda i: (i, 0)),
          pl.BlockSpec((rb, L), lambda i: (i, 0)),
          full(wox), full(box), full(woy), full(boy), full(wat), full(bat),
          full(grp),
      ],
      out_specs=[row_spec] * 2,
      out_shape=[jax.ShapeDtypeStruct((n // G8, 4, G8, NLANE), jnp.int32),
                 jax.ShapeDtypeStruct((n // G8, 4, G8, NLANE), jnp.float32)],
  )(hid, rpx, rpy, wox, box, woy, boy, wat, bat, grp)


def _sc_body(table, i_all, w_all, out,
             idxv0, idxv1, wv0, wv1, gbuf0, gbuf1, obuf, sem0, sem1, ssem):
  wid = lax.axis_index("s") * NC + lax.axis_index("c")
  idxvs = (idxv0, idxv1)
  wvs = (wv0, wv1)
  gbufs = (gbuf0, gbuf1)
  sems = (sem0, sem1)

  def gather_descs(wave, p, ctor):
    buf, sem = gbufs[wave % 2], sems[wave % 2]
    return [
        ctor(table.at[idxvs[p].at[ci, wave * SUB + k]],
             buf.at[ci * SUB + k], sem)
        for ci in range(4) for k in range(SUB)
    ]

  def fire(wave, p):
    gather_descs(wave, p, pltpu.async_copy)

  def wait_wave(wave, p):
    for d_ in gather_descs(wave, p, pltpu.make_async_copy):
      d_.wait()

  def stage(i, p, ctor):
    gs = jnp.minimum(wid + NW * i, NGRP - 1)
    return [ctor(i_all.at[gs], idxvs[p], ssem),
            ctor(w_all.at[gs], wvs[p], ssem)]

  def accum_wave(wave, p):
    buf = gbufs[wave % 2]

    def accum(kh, c2):
      k = kh >> 3
      h16 = (kh & 7) * 16
      acc0 = jnp.zeros((16,), jnp.float32)
      acc1 = jnp.zeros((16,), jnp.float32)
      for c in range(4):
        j = c * SUB + k
        wrow = wvs[p][c, wave * SUB + k, pl.ds(h16, 16)]
        for lp in range(16):
          wgt = wrow[lp]
          wrd = buf[j, h16 + lp, 0:16]
          lo = plsc.bitcast(wrd << 16, jnp.float32)
          hi = plsc.bitcast(wrd & jnp.uint32(0xFFFF0000), jnp.float32)
          acc0 = acc0 + lo * wgt
          acc1 = acc1 + hi * wgt
      obuf[wave * (SUB * H) + kh, 0:16] = acc0
      obuf[wave * (SUB * H) + kh, 16:32] = acc1
      return c2

    lax.fori_loop(0, SUB * H, accum, 0)

  def process_group(i, p, last):
    g = wid + NW * i
    fire(1, p)
    wait_wave(0, p)
    accum_wave(0, p)
    if not last:
      stage(i + 1, 1 - p, pltpu.async_copy)
    wait_wave(1, p)
    if not last:
      for d_ in stage(i + 1, 1 - p, pltpu.make_async_copy):
        d_.wait()
      fire(0, 1 - p)
    accum_wave(1, p)

    @pl.when(g < NGRP)
    def _():
      pltpu.sync_copy(obuf, out.at[pl.ds(g * (G8 * H), G8 * H)])

  for d_ in stage(0, 0, pltpu.async_copy):
    d_.wait()
  fire(0, 0)

  def pair(i2, carry):
    process_group(2 * i2, 0, False)
    process_group(2 * i2 + 1, 1, False)
    return carry

  lax.fori_loop(0, (NITER - 1) // 2, pair, 0)
  process_group(NITER - 1, 0, True)


def _sc_gather(table, i_all, w_all):
  mesh = plsc.VectorSubcoreMesh(core_axis_name="c", subcore_axis_name="s",
                                num_cores=NC, num_subcores=NS)
  fn = pl.kernel(
      _sc_body,
      out_type=jax.ShapeDtypeStruct((BQ * H, Dh), jnp.float32),
      mesh=mesh,
      scratch_types=[
          pltpu.VMEM((4, G8, NLANE), jnp.int32),
          pltpu.VMEM((4, G8, NLANE), jnp.int32),
          pltpu.VMEM((4, G8, NLANE), jnp.float32),
          pltpu.VMEM((4, G8, NLANE), jnp.float32),
          pltpu.VMEM((NGATH, NLANE, Dh // 2), jnp.uint32),
          pltpu.VMEM((NGATH, NLANE, Dh // 2), jnp.uint32),
          pltpu.VMEM((G8 * H, Dh), jnp.float32),
          pltpu.SemaphoreType.DMA,
          pltpu.SemaphoreType.DMA,
          pltpu.SemaphoreType.DMA,
      ],
      compiler_params=pltpu.CompilerParams(use_tc_tiling_on_sc=False,
                                           needs_layout_passes=False),
  )
  return fn(table, i_all, w_all)


def kernel(hidden_states, encoder_hidden_states, reference_points,
           spatial_shapes, level_start_index, W_value, b_value, W_off, b_off,
           W_attn, b_attn, W_out, b_out):
  enc = encoder_hidden_states.reshape(B * S, D)
  cols = np.arange(D).reshape(H, 2, 16)
  lo_cols = cols[:, 0, :].reshape(-1)
  hi_cols = cols[:, 1, :].reshape(-1)
  packed = _value_pack(enc, W_value[:, lo_cols], W_value[:, hi_cols],
                       b_value[lo_cols].reshape(1, -1),
                       b_value[hi_cols].reshape(1, -1), rb=2176)
  table = packed.reshape(B * S * H, Dh // 2)

  hid = hidden_states.reshape(BQ, D)
  rpx = reference_points[..., 0].reshape(BQ, L)
  rpy = reference_points[..., 1].reshape(BQ, L)
  wox = W_off[:, 0::2]
  woy = W_off[:, 1::2]
  box = b_off[0::2].reshape(1, NLANE)
  boy = b_off[1::2].reshape(1, NLANE)
  bat = b_attn.reshape(1, NLANE)

  i_all, w_all = _corner_kernel(
      hid, rpx, rpy, wox, box, woy, boy, W_attn, bat, rb=720)

  sampled = _sc_gather(table, i_all, w_all)
  out = _linear(sampled.reshape(BQ, D), W_out, b_out, rb=480)
  return out.reshape(B, Q, D)
